# baseline (speedup 1.0000x reference)
import functools

import jax
import jax.numpy as jnp
from jax import lax
from jax.experimental import pallas as pl
from jax.experimental.pallas import tpu as pltpu
from jax.experimental.pallas import tpu_sc as plsc

NC = 2
NS = 16
NW = NC * NS

R = 128
NBUF = 8
DEPTH = 4


@functools.partial(jax.jit, static_argnums=(2, 3))
def _sc_gather(idx, table, n, d):
    rows_per_worker = n // NW
    phases = rows_per_worker // R
    outer = phases // NBUF
    assert n % (NW * R) == 0 and phases % NBUF == 0 and outer >= 2
    mesh = plsc.VectorSubcoreMesh(
        core_axis_name="c", subcore_axis_name="s",
        num_cores=NC, num_subcores=NS)

    @functools.partial(
        pl.kernel,
        out_type=jax.ShapeDtypeStruct((n, d), jnp.float32),
        mesh=mesh,
        scratch_types=[
            pltpu.VMEM((NBUF, R), jnp.int32),
            pltpu.VMEM((NBUF, R, d), jnp.float32),
            pltpu.VMEM_SHARED((48, 64), jnp.float32),
        ] + [pltpu.SemaphoreType.DMA] * (3 * NBUF),
        compiler_params=pltpu.CompilerParams(use_tc_tiling_on_sc=False),
    )
    def k(idx_hbm, table_hbm, out_hbm, idx_v, rows_v, table_sh, *all_sems):
        semi = all_sems[0:NBUF]
        semg = all_sems[NBUF:2 * NBUF]
        sems = all_sems[2 * NBUF:3 * NBUF]
        wid = lax.axis_index("s") * NC + lax.axis_index("c")
        wbase = wid * rows_per_worker

        def start_idx(p, slot):
            pltpu.async_copy(
                idx_hbm.at[pl.ds(wbase + p * R, R)], idx_v.at[slot],
                semi[slot])

        def wait_idx(slot):
            pltpu.make_async_copy(
                idx_hbm.at[pl.ds(wbase, R)], idx_v.at[slot],
                semi[slot]).wait()

        def start_gather(slot):
            pltpu.async_copy(
                table_sh.at[idx_v.at[slot]], rows_v.at[slot],
                semg[slot])

        def drain_gather(slot):
            pltpu.make_async_copy(
                out_hbm.at[pl.ds(wbase, R)], rows_v.at[slot],
                semg[slot]).wait()

        def start_store(p, slot):
            pltpu.async_copy(
                rows_v.at[slot], out_hbm.at[pl.ds(wbase + p * R, R)],
                sems[slot])

        def wait_store(slot):
            pltpu.make_async_copy(
                rows_v.at[slot], out_hbm.at[pl.ds(wbase, R)],
                sems[slot]).wait()

        @pl.when(lax.axis_index("s") == 0)
        def _():
            pltpu.sync_copy(table_hbm, table_sh)
        plsc.subcore_barrier()

        for i in range(NBUF):
            start_idx(i, i)

        def body(g, carry):
            for i in range(NBUF):
                p = g * NBUF + i
                jslot = (i + DEPTH) % NBUF

                @pl.when(g > 0)
                def _():
                    wait_store(i)

                wait_idx(i)
                start_gather(i)

                if i >= DEPTH:
                    drain_gather(jslot)
                    start_store(p - DEPTH, jslot)
                    @pl.when(g < outer - 1)
                    def _():
                        start_idx(p + DEPTH, jslot)
                else:
                    @pl.when(g > 0)
                    def _():
                        drain_gather(jslot)
                        start_store(p - DEPTH, jslot)
                        start_idx(p + DEPTH, jslot)
            return carry

        lax.fori_loop(0, outer, body, 0, unroll=False)

        last = outer * NBUF
        for i in range(DEPTH):
            slot = (i + DEPTH) % NBUF
            drain_gather(slot)
            start_store(last - DEPTH + i, slot)
        for i in range(NBUF):
            wait_store(i)

    return k(idx, table)


def kernel(timedelta, table):
    b, h = timedelta.shape
    v, d = table.shape
    n = b * h
    idx = timedelta.astype(jnp.int32).reshape(n)
    out = _sc_gather(idx, table, n, d)
    return out.reshape(b, h, d)


# speedup vs baseline: 1.4202x; 1.4202x over previous
import functools

import jax
import jax.numpy as jnp
from jax import lax
from jax.experimental.layout import Layout, with_layout_constraint
from jax.experimental import pallas as pl
from jax.experimental.pallas import tpu as pltpu
from jax.experimental.pallas import tpu_sc as plsc

NC = 2
NS = 16
NW = NC * NS

R = 128
NBUF = 8
DEPTH = 4


@functools.partial(jax.jit, static_argnums=(2, 3))
def _sc_gather(idx, table, n, d):
    rows_per_worker = n // NW
    phases = rows_per_worker // R
    outer = phases // NBUF
    assert n % (NW * R) == 0 and phases % NBUF == 0 and outer >= 2
    mesh = plsc.VectorSubcoreMesh(
        core_axis_name="c", subcore_axis_name="s",
        num_cores=NC, num_subcores=NS)

    @functools.partial(
        pl.kernel,
        out_type=jax.ShapeDtypeStruct((n, d), jnp.float32),
        mesh=mesh,
        scratch_types=[
            pltpu.VMEM((NBUF, R), jnp.int32),
            pltpu.VMEM((NBUF, R, d), jnp.float32),
            pltpu.VMEM_SHARED((48, 64), jnp.float32),
        ] + [pltpu.SemaphoreType.DMA] * (3 * NBUF),
        compiler_params=pltpu.CompilerParams(use_tc_tiling_on_sc=False),
    )
    def k(idx_hbm, table_hbm, out_hbm, idx_v, rows_v, table_sh, *all_sems):
        semi = all_sems[0:NBUF]
        semg = all_sems[NBUF:2 * NBUF]
        sems = all_sems[2 * NBUF:3 * NBUF]
        wid = lax.axis_index("s") * NC + lax.axis_index("c")
        wbase = wid * rows_per_worker

        def start_idx(p, slot):
            pltpu.async_copy(
                idx_hbm.at[pl.ds(wbase + p * R, R)], idx_v.at[slot],
                semi[slot])

        def wait_idx(slot):
            pltpu.make_async_copy(
                idx_hbm.at[pl.ds(wbase, R)], idx_v.at[slot],
                semi[slot]).wait()

        def start_gather(slot):
            pltpu.async_copy(
                table_sh.at[idx_v.at[slot]], rows_v.at[slot],
                semg[slot])

        def drain_gather(slot):
            pltpu.make_async_copy(
                out_hbm.at[pl.ds(wbase, R)], rows_v.at[slot],
                semg[slot]).wait()

        def start_store(p, slot):
            pltpu.async_copy(
                rows_v.at[slot], out_hbm.at[pl.ds(wbase + p * R, R)],
                sems[slot])

        def wait_store(slot):
            pltpu.make_async_copy(
                rows_v.at[slot], out_hbm.at[pl.ds(wbase, R)],
                sems[slot]).wait()

        @pl.when(lax.axis_index("s") == 0)
        def _():
            pltpu.sync_copy(table_hbm, table_sh)
        plsc.subcore_barrier()

        for i in range(NBUF):
            start_idx(i, i)

        def body(g, carry):
            for i in range(NBUF):
                p = g * NBUF + i
                jslot = (i + DEPTH) % NBUF

                @pl.when(g > 0)
                def _():
                    wait_store(i)

                wait_idx(i)
                start_gather(i)

                if i >= DEPTH:
                    drain_gather(jslot)
                    start_store(p - DEPTH, jslot)
                    @pl.when(g < outer - 1)
                    def _():
                        start_idx(p + DEPTH, jslot)
                else:
                    @pl.when(g > 0)
                    def _():
                        drain_gather(jslot)
                        start_store(p - DEPTH, jslot)
                        start_idx(p + DEPTH, jslot)
            return carry

        lax.fori_loop(0, outer, body, 0, unroll=False)

        last = outer * NBUF
        for i in range(DEPTH):
            slot = (i + DEPTH) % NBUF
            drain_gather(slot)
            start_store(last - DEPTH + i, slot)
        for i in range(NBUF):
            wait_store(i)

    return k(idx, table)


def kernel(timedelta, table):
    b, h = timedelta.shape
    v, d = table.shape
    n = b * h
    idx = timedelta.astype(jnp.int32).reshape(n)
    out = _sc_gather(idx, table, n, d)
    return with_layout_constraint(out.reshape(b, h, d), Layout((0, 1, 2)))
